# baseline (speedup 1.0000x reference)
import jax
import jax.numpy as jnp
from jax import lax
from jax.experimental import pallas as pl
from jax.experimental.pallas import tpu as pltpu
from jax.experimental.pallas import tpu_sc as plsc

E = 16
F = 26
D = F * E
DP = 512
V = 100000
B = 4096
NC, NS = 2, 16
NW = NC * NS
SPW = B // NW
SPC = 4
NCHUNK = SPW // SPC
RPC = SPC * F

VC = 2048
NVB = -(-V // VC)

_MESH = plsc.VectorSubcoreMesh(
    core_axis_name="c", subcore_axis_name="s", num_cores=NC, num_subcores=NS)


def _relayout_body(in_ref, w1_ref, out_ref):
    x = in_ref[...]
    x2 = x.reshape(D, VC)
    w1r = w1_ref[...].reshape(1, VC)
    xp = jnp.concatenate(
        [x2, w1r, jnp.zeros((DP - D - 1, VC), jnp.float32)], axis=0)
    y = jnp.transpose(xp, (1, 0))
    out_ref[...] = y.reshape(VC * DP // 128, 128)


def _relayout(emb_t, w1_flat):
    return pl.pallas_call(
        _relayout_body,
        grid=(NVB,),
        in_specs=[pl.BlockSpec((F, E, VC), lambda i: (0, 0, i)),
                  pl.BlockSpec((VC,), lambda i: (i,))],
        out_specs=pl.BlockSpec((VC * DP // 128, 128), lambda i: (i, 0)),
        out_shape=jax.ShapeDtypeStruct((V * DP // 128, 128), jnp.float32),
    )(emb_t, w1_flat)


def _ffm_body(emb_hbm, idx_hbm, val_hbm, out_hbm,
              idx_v, val_v, rows_v, res_v, sem_r0, sem_r1):
    wid = lax.axis_index("s") * NC + lax.axis_index("c")
    sbase = wid * SPW
    rbase = sbase * F
    pltpu.sync_copy(idx_hbm.at[pl.ds(rbase, SPW * F)], idx_v)
    pltpu.sync_copy(val_hbm.at[pl.ds(rbase, SPW * F)],
                    val_v.at[pl.ds(0, SPW * F)])
    lane = lax.iota(jnp.int32, E)
    tail_mask = lane < (F - E)
    sems_r = (sem_r0, sem_r1)
    col_d = jnp.full((E,), D, jnp.int32)

    def copies(ck, par):
        isl = idx_v.at[pl.ds(ck * RPC, RPC)]
        return (pltpu.make_async_copy(emb_hbm.at[isl],
                                      rows_v.at[par], sems_r[par]),)

    for c in copies(0, 0):
        c.start()

    def pair_body(m, resvec):
        for par in (0, 1):
            ck = 2 * m + par

            @pl.when(ck + 1 < NCHUNK)
            def _():
                for c in copies(ck + 1, 1 - par):
                    c.start()

            for c in copies(ck, par):
                c.wait()

            def samp_body(sl, rv):
                voff = ck * RPC + sl * F
                v0 = val_v[pl.ds(voff, E)]
                v1 = val_v[pl.ds(voff + E, E)]
                xs = ([v0[i] for i in range(E)]
                      + [v1[i] for i in range(F - E)])
                r0 = sl * F
                par_vec = jnp.full((E,), par, jnp.int32)
                u0 = plsc.load_gather(rows_v, [par_vec, r0 + lane, col_d])
                u1 = plsc.load_gather(
                    rows_v, [par_vec, r0 + E + lane, col_d], mask=tail_mask)
                fo = jnp.sum(v0 * u0) + jnp.sum(
                    jnp.where(tail_mask, v1 * u1, jnp.float32(0.0)))
                acc = jnp.zeros((E,), jnp.float32)
                for i in range(F):
                    for j in range(i + 1, F):
                        vi = rows_v[par, r0 + i, pl.ds(j * E, E)]
                        vj = rows_v[par, r0 + j, pl.ds(i * E, E)]
                        acc = acc + (xs[i] * xs[j]) * (vi * vj)
                total = jnp.sum(acc) + fo
                gs = ck * SPC + sl
                return jnp.where(lane == gs % E, total, rv)

            resvec = lax.fori_loop(0, SPC, samp_body, resvec)

            @pl.when((ck % (E // SPC)) == (E // SPC - 1))
            def _():
                res_v[pl.ds((ck // (E // SPC)) * E, E)] = resvec

        return resvec

    lax.fori_loop(0, NCHUNK // 2, pair_body, jnp.zeros((E,), jnp.float32))
    pltpu.sync_copy(res_v, out_hbm.at[pl.ds(sbase, SPW)])


@jax.jit
def _ffm_call(table_pad, idx_flat, val_flat):
    run = pl.kernel(
        _ffm_body,
        out_type=jax.ShapeDtypeStruct((B,), jnp.float32),
        mesh=_MESH,
        compiler_params=pltpu.CompilerParams(
            needs_layout_passes=False, use_tc_tiling_on_sc=False),
        scratch_types=[
            pltpu.VMEM((SPW * F,), jnp.int32),
            pltpu.VMEM((SPW * F + E,), jnp.float32),
            pltpu.VMEM((2, RPC, DP), jnp.float32),
            pltpu.VMEM((SPW,), jnp.float32),
            pltpu.SemaphoreType.DMA,
            pltpu.SemaphoreType.DMA,
        ],
    )
    return run(table_pad, idx_flat, val_flat)


def kernel(feature_idx, feature_values, feature_embeddings,
           weights_first_order, fm_bias):
    idx_flat = feature_idx.reshape(-1).astype(jnp.int32)
    val_flat = feature_values.reshape(-1)
    emb_t = jnp.transpose(feature_embeddings, (1, 2, 0))
    table_pad = _relayout(emb_t, weights_first_order.reshape(-1)).reshape(V, DP)
    out = _ffm_call(table_pad, idx_flat, val_flat)
    return out.reshape(B, 1) + fm_bias


# speedup vs baseline: 1.1712x; 1.1712x over previous
import jax
import jax.numpy as jnp
from jax import lax
from jax.experimental import pallas as pl
from jax.experimental.pallas import tpu as pltpu
from jax.experimental.pallas import tpu_sc as plsc

E = 16
F = 26
D = F * E
DP = 512
V = 100000
B = 4096
NC, NS = 2, 16
NW = NC * NS
SPW = B // NW
SPC = 4
NCHUNK = SPW // SPC
RPC = SPC * F

VC = 2048
NVB = -(-V // VC)

_MESH = plsc.VectorSubcoreMesh(
    core_axis_name="c", subcore_axis_name="s", num_cores=NC, num_subcores=NS)


DW = DP // 2


def _relayout_body(in_ref, w1_ref, out_ref):
    x = in_ref[...]
    x2 = x.reshape(D, VC)
    w1r = w1_ref[...].reshape(1, VC)
    xp = jnp.concatenate(
        [x2, w1r, jnp.zeros((DP - D - 1, VC), jnp.float32)], axis=0)
    xp4 = xp.reshape(E, 2, E, VC)
    xe = xp4[:, 0, :, :].reshape(DW, VC)
    xo = xp4[:, 1, :, :].reshape(DW, VC)
    ye = jnp.transpose(xe, (1, 0))
    yo = jnp.transpose(xo, (1, 0))

    def rtne(u):
        return u + 0x7FFF + ((u >> 16) & 1)

    ua = rtne(jax.lax.bitcast_convert_type(ye, jnp.int32))
    ub = rtne(jax.lax.bitcast_convert_type(yo, jnp.int32))
    packed = ((ua >> 16) & 0xFFFF) | (ub & jnp.int32(-65536))
    out_ref[...] = packed.reshape(VC * DW // 128, 128)


def _relayout(emb_t, w1_flat):
    return pl.pallas_call(
        _relayout_body,
        grid=(NVB,),
        in_specs=[pl.BlockSpec((F, E, VC), lambda i: (0, 0, i)),
                  pl.BlockSpec((VC,), lambda i: (i,))],
        out_specs=pl.BlockSpec((VC * DW // 128, 128), lambda i: (i, 0)),
        out_shape=jax.ShapeDtypeStruct((V * DW // 128, 128), jnp.int32),
    )(emb_t, w1_flat)


def _ffm_body(emb_hbm, idx_hbm, val_hbm, out_hbm,
              idx_v, val_v, rows_v, res_v, sem_r0, sem_r1):
    wid = lax.axis_index("s") * NC + lax.axis_index("c")
    sbase = wid * SPW
    rbase = sbase * F
    pltpu.sync_copy(idx_hbm.at[pl.ds(rbase, SPW * F)], idx_v)
    pltpu.sync_copy(val_hbm.at[pl.ds(rbase, SPW * F)],
                    val_v.at[pl.ds(0, SPW * F)])
    lane = lax.iota(jnp.int32, E)
    tail_mask = lane < (F - E)
    sems_r = (sem_r0, sem_r1)
    col_d = jnp.full((E,), D // 2, jnp.int32)
    himask = jnp.full((E,), -65536, jnp.int32)

    def lo_f32(u):
        return jax.lax.bitcast_convert_type(u << 16, jnp.float32)

    def hi_f32(u):
        return jax.lax.bitcast_convert_type(u & himask, jnp.float32)

    def copies(ck, par):
        isl = idx_v.at[pl.ds(ck * RPC, RPC)]
        return (pltpu.make_async_copy(emb_hbm.at[isl],
                                      rows_v.at[par], sems_r[par]),)

    for c in copies(0, 0):
        c.start()

    def pair_body(m, resvec):
        for par in (0, 1):
            ck = 2 * m + par

            @pl.when(ck + 1 < NCHUNK)
            def _():
                for c in copies(ck + 1, 1 - par):
                    c.start()

            for c in copies(ck, par):
                c.wait()

            def samp_body(sl, rv):
                voff = ck * RPC + sl * F
                v0 = val_v[pl.ds(voff, E)]
                v1 = val_v[pl.ds(voff + E, E)]
                xs = ([v0[i] for i in range(E)]
                      + [v1[i] for i in range(F - E)])
                r0 = sl * F
                par_vec = jnp.full((E,), par, jnp.int32)
                u0 = plsc.load_gather(rows_v, [par_vec, r0 + lane, col_d])
                u1 = plsc.load_gather(
                    rows_v, [par_vec, r0 + E + lane, col_d], mask=tail_mask)
                fo = jnp.sum(v0 * lo_f32(u0)) + jnp.sum(
                    jnp.where(tail_mask, v1 * lo_f32(u1), jnp.float32(0.0)))
                acc = jnp.zeros((E,), jnp.float32)
                for i in range(F):
                    for a in range((i + 1) // 2, F // 2):
                        ui = rows_v[par, r0 + i, pl.ds(a * E, E)]
                        for b in (0, 1):
                            j = 2 * a + b
                            if j <= i:
                                continue
                            uj = rows_v[par, r0 + j,
                                        pl.ds((i // 2) * E, E)]
                            vi = lo_f32(ui) if b == 0 else hi_f32(ui)
                            vj = (lo_f32(uj) if i % 2 == 0
                                  else hi_f32(uj))
                            acc = acc + (xs[i] * xs[j]) * (vi * vj)
                total = jnp.sum(acc) + fo
                gs = ck * SPC + sl
                return jnp.where(lane == gs % E, total, rv)

            resvec = lax.fori_loop(0, SPC, samp_body, resvec)

            @pl.when((ck % (E // SPC)) == (E // SPC - 1))
            def _():
                res_v[pl.ds((ck // (E // SPC)) * E, E)] = resvec

        return resvec

    lax.fori_loop(0, NCHUNK // 2, pair_body, jnp.zeros((E,), jnp.float32))
    pltpu.sync_copy(res_v, out_hbm.at[pl.ds(sbase, SPW)])


@jax.jit
def _ffm_call(table_pad, idx_flat, val_flat):
    run = pl.kernel(
        _ffm_body,
        out_type=jax.ShapeDtypeStruct((B,), jnp.float32),
        mesh=_MESH,
        compiler_params=pltpu.CompilerParams(
            needs_layout_passes=False, use_tc_tiling_on_sc=False),
        scratch_types=[
            pltpu.VMEM((SPW * F,), jnp.int32),
            pltpu.VMEM((SPW * F + E,), jnp.float32),
            pltpu.VMEM((2, RPC, DW), jnp.int32),
            pltpu.VMEM((SPW,), jnp.float32),
            pltpu.SemaphoreType.DMA,
            pltpu.SemaphoreType.DMA,
        ],
    )
    return run(table_pad, idx_flat, val_flat)


def kernel(feature_idx, feature_values, feature_embeddings,
           weights_first_order, fm_bias):
    idx_flat = feature_idx.reshape(-1).astype(jnp.int32)
    val_flat = feature_values.reshape(-1)
    emb_t = jnp.transpose(feature_embeddings, (1, 2, 0))
    table_pack = _relayout(
        emb_t, weights_first_order.reshape(-1)).reshape(V, DW)
    out = _ffm_call(table_pack, idx_flat, val_flat)
    return out.reshape(B, 1) + fm_bias


# speedup vs baseline: 1.2189x; 1.0408x over previous
import jax
import jax.numpy as jnp
from jax import lax
from jax.experimental import pallas as pl
from jax.experimental.pallas import tpu as pltpu
from jax.experimental.pallas import tpu_sc as plsc

E = 16
F = 26
D = F * E
DP = 512
V = 100000
B = 4096
NC, NS = 2, 16
NW = NC * NS
SPW = B // NW
SPC = 4
NCHUNK = SPW // SPC
RPC = SPC * F

VC = 4096
NVB = -(-V // VC)

_MESH = plsc.VectorSubcoreMesh(
    core_axis_name="c", subcore_axis_name="s", num_cores=NC, num_subcores=NS)


DW = DP // 2


def _relayout_body(in_ref, w1_ref, out_ref):
    x = in_ref[...]
    x2 = x.reshape(D, VC)
    w1r = w1_ref[...].reshape(1, VC)
    xp = jnp.concatenate(
        [x2, w1r, jnp.zeros((DP - D - 1, VC), jnp.float32)], axis=0)
    xp4 = xp.reshape(E, 2, E, VC)
    xe = xp4[:, 0, :, :].reshape(DW, VC)
    xo = xp4[:, 1, :, :].reshape(DW, VC)
    ye = jnp.transpose(xe, (1, 0))
    yo = jnp.transpose(xo, (1, 0))

    def rtne(u):
        return u + 0x7FFF + ((u >> 16) & 1)

    ua = rtne(jax.lax.bitcast_convert_type(ye, jnp.int32))
    ub = rtne(jax.lax.bitcast_convert_type(yo, jnp.int32))
    packed = ((ua >> 16) & 0xFFFF) | (ub & jnp.int32(-65536))
    out_ref[...] = packed.reshape(VC * DW // 128, 128)


def _relayout(emb_t, w1_flat):
    return pl.pallas_call(
        _relayout_body,
        grid=(NVB,),
        in_specs=[pl.BlockSpec((F, E, VC), lambda i: (0, 0, i)),
                  pl.BlockSpec((VC,), lambda i: (i,))],
        out_specs=pl.BlockSpec((VC * DW // 128, 128), lambda i: (i, 0)),
        out_shape=jax.ShapeDtypeStruct((V * DW // 128, 128), jnp.int32),
    )(emb_t, w1_flat)


def _ffm_body(emb_hbm, idx_hbm, val_hbm, out_hbm,
              idx_v, val_v, rows_v, res_v, sem_r0, sem_r1):
    wid = lax.axis_index("s") * NC + lax.axis_index("c")
    sbase = wid * SPW
    rbase = sbase * F
    pltpu.sync_copy(idx_hbm.at[pl.ds(rbase, SPW * F)], idx_v)
    pltpu.sync_copy(val_hbm.at[pl.ds(rbase, SPW * F)],
                    val_v.at[pl.ds(0, SPW * F)])
    lane = lax.iota(jnp.int32, E)
    tail_mask = lane < (F - E)
    sems_r = (sem_r0, sem_r1)
    col_d = jnp.full((E,), D // 2, jnp.int32)
    himask = jnp.full((E,), -65536, jnp.int32)

    def lo_f32(u):
        return jax.lax.bitcast_convert_type(u << 16, jnp.float32)

    def hi_f32(u):
        return jax.lax.bitcast_convert_type(u & himask, jnp.float32)

    def copies(ck, par):
        isl = idx_v.at[pl.ds(ck * RPC, RPC)]
        return (pltpu.make_async_copy(emb_hbm.at[isl],
                                      rows_v.at[par], sems_r[par]),)

    for c in copies(0, 0):
        c.start()

    def pair_body(m, resvec):
        for par in (0, 1):
            ck = 2 * m + par

            @pl.when(ck + 1 < NCHUNK)
            def _():
                for c in copies(ck + 1, 1 - par):
                    c.start()

            for c in copies(ck, par):
                c.wait()

            def samp_body(sl, rv):
                voff = ck * RPC + sl * F
                v0 = val_v[pl.ds(voff, E)]
                v1 = val_v[pl.ds(voff + E, E)]
                xs = ([v0[i] for i in range(E)]
                      + [v1[i] for i in range(F - E)])
                r0 = sl * F
                par_vec = jnp.full((E,), par, jnp.int32)
                u0 = plsc.load_gather(rows_v, [par_vec, r0 + lane, col_d])
                u1 = plsc.load_gather(
                    rows_v, [par_vec, r0 + E + lane, col_d], mask=tail_mask)
                fo = jnp.sum(v0 * lo_f32(u0)) + jnp.sum(
                    jnp.where(tail_mask, v1 * lo_f32(u1), jnp.float32(0.0)))
                acc = jnp.zeros((E,), jnp.float32)
                for i in range(F):
                    for a in range((i + 1) // 2, F // 2):
                        ui = rows_v[par, r0 + i, pl.ds(a * E, E)]
                        for b in (0, 1):
                            j = 2 * a + b
                            if j <= i:
                                continue
                            uj = rows_v[par, r0 + j,
                                        pl.ds((i // 2) * E, E)]
                            vi = lo_f32(ui) if b == 0 else hi_f32(ui)
                            vj = (lo_f32(uj) if i % 2 == 0
                                  else hi_f32(uj))
                            acc = acc + (xs[i] * xs[j]) * (vi * vj)
                total = jnp.sum(acc) + fo
                gs = ck * SPC + sl
                return jnp.where(lane == gs % E, total, rv)

            resvec = lax.fori_loop(0, SPC, samp_body, resvec)

            @pl.when((ck % (E // SPC)) == (E // SPC - 1))
            def _():
                res_v[pl.ds((ck // (E // SPC)) * E, E)] = resvec

        return resvec

    lax.fori_loop(0, NCHUNK // 2, pair_body, jnp.zeros((E,), jnp.float32))
    pltpu.sync_copy(res_v, out_hbm.at[pl.ds(sbase, SPW)])


@jax.jit
def _ffm_call(table_pad, idx_flat, val_flat):
    run = pl.kernel(
        _ffm_body,
        out_type=jax.ShapeDtypeStruct((B,), jnp.float32),
        mesh=_MESH,
        compiler_params=pltpu.CompilerParams(
            needs_layout_passes=False, use_tc_tiling_on_sc=False),
        scratch_types=[
            pltpu.VMEM((SPW * F,), jnp.int32),
            pltpu.VMEM((SPW * F + E,), jnp.float32),
            pltpu.VMEM((2, RPC, DW), jnp.int32),
            pltpu.VMEM((SPW,), jnp.float32),
            pltpu.SemaphoreType.DMA,
            pltpu.SemaphoreType.DMA,
        ],
    )
    return run(table_pad, idx_flat, val_flat)


def kernel(feature_idx, feature_values, feature_embeddings,
           weights_first_order, fm_bias):
    idx_flat = feature_idx.reshape(-1).astype(jnp.int32)
    val_flat = feature_values.reshape(-1)
    emb_t = jnp.transpose(feature_embeddings, (1, 2, 0))
    table_pack = _relayout(
        emb_t, weights_first_order.reshape(-1)).reshape(V, DW)
    out = _ffm_call(table_pack, idx_flat, val_flat)
    return out.reshape(B, 1) + fm_bias


# speedup vs baseline: 1.3020x; 1.0682x over previous
import jax
import jax.numpy as jnp
from jax import lax
from jax.experimental import pallas as pl
from jax.experimental.pallas import tpu as pltpu
from jax.experimental.pallas import tpu_sc as plsc

E = 16
F = 26
D = F * E
DP = 512
V = 100000
B = 4096
NC, NS = 2, 16
NW = NC * NS
SPW = B // NW
SPC = 4
NCHUNK = SPW // SPC
RPC = SPC * F

VC = 4096
NVB = -(-V // VC)

_MESH = plsc.VectorSubcoreMesh(
    core_axis_name="c", subcore_axis_name="s", num_cores=NC, num_subcores=NS)


DW = DP // 2


def _relayout_body(in_ref, w1_ref, out_ref):
    x = in_ref[...]
    x2 = x.reshape(D, VC)
    w1r = w1_ref[...].reshape(1, VC)
    xp = jnp.concatenate(
        [x2, w1r, jnp.zeros((DP - D - 1, VC), jnp.float32)], axis=0)
    xp4 = xp.reshape(E, 2, E, VC)
    xe = xp4[:, 0, :, :].reshape(DW, VC)
    xo = xp4[:, 1, :, :].reshape(DW, VC)
    ye = jnp.transpose(xe, (1, 0))
    yo = jnp.transpose(xo, (1, 0))

    def rtne(u):
        return u + 0x7FFF + ((u >> 16) & 1)

    ua = rtne(jax.lax.bitcast_convert_type(ye, jnp.int32))
    ub = rtne(jax.lax.bitcast_convert_type(yo, jnp.int32))
    packed = ((ua >> 16) & 0xFFFF) | (ub & jnp.int32(-65536))
    out_ref[...] = packed.reshape(VC * DW // 128, 128)


def _relayout(emb_t, w1_flat):
    return pl.pallas_call(
        _relayout_body,
        grid=(NVB,),
        in_specs=[pl.BlockSpec((F, E, VC), lambda i: (0, 0, i)),
                  pl.BlockSpec((VC,), lambda i: (i,))],
        out_specs=pl.BlockSpec((VC * DW // 128, 128), lambda i: (i, 0)),
        out_shape=jax.ShapeDtypeStruct((V * DW // 128, 128), jnp.int32),
    )(emb_t, w1_flat)


def _ffm_body(emb_hbm, idx_hbm, val_hbm, out_hbm,
              idx_v, val_v, rows_v, res_v, sem_r0, sem_r1):
    wid = lax.axis_index("s") * NC + lax.axis_index("c")
    sbase = wid * SPW
    rbase = sbase * F
    pltpu.sync_copy(idx_hbm.at[pl.ds(rbase, SPW * F)], idx_v)
    pltpu.sync_copy(val_hbm.at[pl.ds(rbase, SPW * F)],
                    val_v.at[pl.ds(0, SPW * F)])
    lane = lax.iota(jnp.int32, E)
    tail_mask = lane < (F - E)
    sems_r = (sem_r0, sem_r1)
    col_d = jnp.full((E,), D // 2, jnp.int32)
    himask = jnp.full((E,), -65536, jnp.int32)

    def lo_f32(u):
        return jax.lax.bitcast_convert_type(u << 16, jnp.float32)

    def hi_f32(u):
        return jax.lax.bitcast_convert_type(u & himask, jnp.float32)

    def copies(ck, par):
        isl = idx_v.at[pl.ds(ck * RPC, RPC)]
        return (pltpu.make_async_copy(emb_hbm.at[isl],
                                      rows_v.at[par], sems_r[par]),)

    for c in copies(0, 0):
        c.start()

    def pair_body(m, resvec):
        for par in (0, 1):
            ck = 2 * m + par

            @pl.when(ck + 1 < NCHUNK)
            def _():
                for c in copies(ck + 1, 1 - par):
                    c.start()

            for c in copies(ck, par):
                c.wait()

            def samp_body(sl, rv):
                voff = ck * RPC + sl * F
                v0 = val_v[pl.ds(voff, E)]
                v1 = val_v[pl.ds(voff + E, E)]
                xs = ([v0[i] for i in range(E)]
                      + [v1[i] for i in range(F - E)])
                r0 = sl * F
                par_vec = jnp.full((E,), par, jnp.int32)
                u0 = plsc.load_gather(rows_v, [par_vec, r0 + lane, col_d])
                u1 = plsc.load_gather(
                    rows_v, [par_vec, r0 + E + lane, col_d], mask=tail_mask)
                fo = jnp.sum(v0 * lo_f32(u0)) + jnp.sum(
                    jnp.where(tail_mask, v1 * lo_f32(u1), jnp.float32(0.0)))
                accs = [jnp.zeros((E,), jnp.float32) for _ in range(4)]
                p = 0
                for i in range(F):
                    for a in range((i + 1) // 2, F // 2):
                        ui = rows_v[par, r0 + i, pl.ds(a * E, E)]
                        for b in (0, 1):
                            j = 2 * a + b
                            if j <= i:
                                continue
                            uj = rows_v[par, r0 + j,
                                        pl.ds((i // 2) * E, E)]
                            vi = lo_f32(ui) if b == 0 else hi_f32(ui)
                            vj = (lo_f32(uj) if i % 2 == 0
                                  else hi_f32(uj))
                            accs[p] = accs[p] + (xs[i] * xs[j]) * (vi * vj)
                            p = (p + 1) % 4
                total = jnp.sum((accs[0] + accs[1]) + (accs[2] + accs[3])) + fo
                gs = ck * SPC + sl
                return jnp.where(lane == gs % E, total, rv)

            resvec = lax.fori_loop(0, SPC, samp_body, resvec)

            @pl.when((ck % (E // SPC)) == (E // SPC - 1))
            def _():
                res_v[pl.ds((ck // (E // SPC)) * E, E)] = resvec

        return resvec

    lax.fori_loop(0, NCHUNK // 2, pair_body, jnp.zeros((E,), jnp.float32))
    pltpu.sync_copy(res_v, out_hbm.at[pl.ds(sbase, SPW)])


@jax.jit
def _ffm_call(table_pad, idx_flat, val_flat):
    run = pl.kernel(
        _ffm_body,
        out_type=jax.ShapeDtypeStruct((B,), jnp.float32),
        mesh=_MESH,
        compiler_params=pltpu.CompilerParams(
            needs_layout_passes=False, use_tc_tiling_on_sc=False),
        scratch_types=[
            pltpu.VMEM((SPW * F,), jnp.int32),
            pltpu.VMEM((SPW * F + E,), jnp.float32),
            pltpu.VMEM((2, RPC, DW), jnp.int32),
            pltpu.VMEM((SPW,), jnp.float32),
            pltpu.SemaphoreType.DMA,
            pltpu.SemaphoreType.DMA,
        ],
    )
    return run(table_pad, idx_flat, val_flat)


def kernel(feature_idx, feature_values, feature_embeddings,
           weights_first_order, fm_bias):
    idx_flat = feature_idx.reshape(-1).astype(jnp.int32)
    val_flat = feature_values.reshape(-1)
    emb_t = jnp.transpose(feature_embeddings, (1, 2, 0))
    table_pack = _relayout(
        emb_t, weights_first_order.reshape(-1)).reshape(V, DW)
    out = _ffm_call(table_pack, idx_flat, val_flat)
    return out.reshape(B, 1) + fm_bias
